# Initial kernel scaffold; baseline (speedup 1.0000x reference)
import jax
import jax.numpy as jnp
from jax import lax
from jax.experimental import pallas as pl
from jax.experimental.pallas import tpu as pltpu
from jax.experimental.pallas import tpu_sc as plsc

M = 100000
D = 128
B = 16384
K = 8
N = B * K
NC = 2
NS = 16
NW = NC * NS

_MESH = dict(core_axis_name="c", subcore_axis_name="s", num_cores=NC,
             num_subcores=NS)

NPW = N // NW
GB = 128
GI = NPW // GB


def _gather_body(val_hbm, state_hbm, idx_hbm, rows_out, svals_out,
                 idx_v, rows_v, svals_v, sem):
    c = lax.axis_index("c")
    s = lax.axis_index("s")
    wid = s * NC + c
    base = wid * NPW
    pltpu.sync_copy(idx_hbm.at[pl.ds(wid * GI, GI)], idx_v)

    def step(j, carry):
        pltpu.async_copy(val_hbm.at[idx_v.at[j]], rows_v, sem).wait()
        pltpu.async_copy(state_hbm.at[idx_v.at[j]], svals_v, sem).wait()
        pltpu.sync_copy(rows_v, rows_out.at[pl.ds(base + j * GB, GB)])
        pltpu.sync_copy(svals_v, svals_out.at[pl.ds(base + j * GB, GB)])
        return carry

    lax.fori_loop(0, GI, step, 0)


def _gather(init_val, init_state, idx2d):
    return pl.kernel(
        _gather_body,
        out_type=[jax.ShapeDtypeStruct((N, D), jnp.float32),
                  jax.ShapeDtypeStruct((N,), jnp.float32)],
        mesh=plsc.VectorSubcoreMesh(**_MESH),
        scratch_types=[pltpu.VMEM((GI, GB), jnp.int32),
                       pltpu.VMEM((GB, D), jnp.float32),
                       pltpu.VMEM((GB,), jnp.float32),
                       pltpu.SemaphoreType.DMA],
    )(init_val, init_state, idx2d)


RB = 512


def _route_body(rows_ref, x_ref, sv_ref, ww_ref, asc_ref):
    xr = x_ref[...]
    r3 = rows_ref[...].reshape(RB, K, D)
    sc = jnp.sum(r3 * xr[:, None, :], axis=-1)
    a = jnp.abs(sc)
    e = jnp.exp(a - jnp.max(a, axis=-1, keepdims=True))
    w = jnp.sign(sc) * e / jnp.sum(e, axis=-1, keepdims=True)
    gate = jax.nn.softplus(sv_ref[...])
    ww_ref[...] = w * gate
    asc_ref[...] = a


def _route(rows, x, sv2d):
    grid = B // RB
    return pl.pallas_call(
        _route_body,
        grid=(grid,),
        in_specs=[pl.BlockSpec((RB * K, D), lambda i: (i, 0)),
                  pl.BlockSpec((RB, D), lambda i: (i, 0)),
                  pl.BlockSpec((RB, K), lambda i: (i, 0))],
        out_specs=[pl.BlockSpec((RB, K), lambda i: (i, 0)),
                   pl.BlockSpec((RB, K), lambda i: (i, 0))],
        out_shape=[jax.ShapeDtypeStruct((B, K), jnp.float32),
                   jax.ShapeDtypeStruct((B, K), jnp.float32)],
    )(rows, x, sv2d)


CH = 10000
P = 5
HB = 512
NHB = B // (NS * HB)
E = HB * K
NV = E // 16
RPT = CH // NS


def _scatter_body(x_hbm, idx_hbm, ww_hbm, asc_hbm, val_hbm, state_hbm,
                  mem_out, state_out,
                  xblk, iblk, wblk, ablk, cli, cu, cww, casc, ustage,
                  chunk, schunk, sem):
    c = lax.axis_index("c")
    s = lax.axis_index("s")
    lanes = lax.iota(jnp.int32, 16)

    def one_pass(p, carry):
        base = (p * NC + c) * CH
        pltpu.sync_copy(val_hbm.at[pl.ds(base + s * RPT, RPT)],
                        chunk.at[pl.ds(s * RPT, RPT)])

        @pl.when(s == 0)
        def _():
            pltpu.sync_copy(state_hbm.at[pl.ds(base, CH)],
                            schunk.at[pl.ds(0, CH)])

        plsc.subcore_barrier()

        for hb in range(NHB):
            b0 = s * (NHB * HB) + hb * HB
            e0 = b0 * K
            pltpu.sync_copy(x_hbm.at[pl.ds(b0, HB)], xblk)
            pltpu.sync_copy(idx_hbm.at[pl.ds(e0, E)], iblk)
            pltpu.sync_copy(ww_hbm.at[pl.ds(e0, E)], wblk)
            pltpu.sync_copy(asc_hbm.at[pl.ds(e0, E)], ablk)

            def compact(v, off):
                iv = iblk[pl.ds(v * 16, 16)]
                li = iv - base
                m = (li >= 0) & (li < CH)
                plsc.store_compressed(cli.at[pl.ds(off, 16)], li, mask=m)
                plsc.store_compressed(cu.at[pl.ds(off, 16)], v * 16 + lanes,
                                      mask=m)
                plsc.store_compressed(cww.at[pl.ds(off, 16)],
                                      wblk[pl.ds(v * 16, 16)], mask=m)
                plsc.store_compressed(casc.at[pl.ds(off, 16)],
                                      ablk[pl.ds(v * 16, 16)], mask=m)
                return off + jnp.sum(jnp.where(m, 1, 0))

            count = lax.fori_loop(0, NV, compact, jnp.int32(0))
            cli[pl.ds(count, 16)] = jnp.full((16,), CH, jnp.int32)
            cu[pl.ds(count, 16)] = jnp.zeros((16,), jnp.int32)
            cww[pl.ds(count, 16)] = jnp.zeros((16,), jnp.float32)
            casc[pl.ds(count, 16)] = jnp.zeros((16,), jnp.float32)
            nbat = (count + 15) // 16

            def batch(t, carry2):
                t0 = t * 16
                for j in range(16):
                    uu = cu[t0 + j]
                    wws = cww[t0 + j]
                    bl = lax.shift_right_logical(uu, 3)
                    for v in range(D // 16):
                        ustage[j, pl.ds(v * 16, 16)] = (
                            xblk[bl, pl.ds(v * 16, 16)] * wws)
                li16 = cli[pl.ds(t0, 16)]
                pltpu.sync_copy(ustage, chunk.at[li16], add=True)
                pltpu.sync_copy(casc.at[pl.ds(t0, 16)], schunk.at[li16],
                                add=True)
                return carry2

            lax.fori_loop(0, nbat, batch, 0)

        plsc.subcore_barrier()
        pltpu.sync_copy(chunk.at[pl.ds(s * RPT, RPT)],
                        mem_out.at[pl.ds(base + s * RPT, RPT)])

        @pl.when(s == 0)
        def _():
            pltpu.sync_copy(schunk.at[pl.ds(0, CH)],
                            state_out.at[pl.ds(base, CH)])

        plsc.subcore_barrier()
        return carry

    lax.fori_loop(0, P, one_pass, 0)


def _scatter(x, idxf, wwf, ascf, init_val, init_state):
    return pl.kernel(
        _scatter_body,
        out_type=[jax.ShapeDtypeStruct((M, D), jnp.float32),
                  jax.ShapeDtypeStruct((M,), jnp.float32)],
        mesh=plsc.VectorSubcoreMesh(**_MESH),
        scratch_types=[pltpu.VMEM((HB, D), jnp.float32),
                       pltpu.VMEM((E,), jnp.int32),
                       pltpu.VMEM((E,), jnp.float32),
                       pltpu.VMEM((E,), jnp.float32),
                       pltpu.VMEM((E + 16,), jnp.int32),
                       pltpu.VMEM((E + 16,), jnp.int32),
                       pltpu.VMEM((E + 16,), jnp.float32),
                       pltpu.VMEM((E + 16,), jnp.float32),
                       pltpu.VMEM((16, D), jnp.float32),
                       pltpu.VMEM_SHARED((CH + 16, D), jnp.float32),
                       pltpu.VMEM_SHARED((CH + 16,), jnp.float32),
                       pltpu.SemaphoreType.DMA],
    )(x, idxf, wwf, ascf, init_val, init_state)


MB = 2500


def _norm_body(mem_ref, g_ref, b_ref, out_ref):
    v = mem_ref[...]
    mu = jnp.mean(v, axis=-1, keepdims=True)
    dlt = v - mu
    var = jnp.mean(dlt * dlt, axis=-1, keepdims=True)
    ln = dlt * lax.rsqrt(var + 1e-5) * g_ref[...] + b_ref[...]
    nrm = jnp.sqrt(jnp.sum(ln * ln, axis=-1, keepdims=True))
    out_ref[...] = ln / (nrm + 1e-6)


def _norm(mem_raw, g2d, b2d):
    return pl.pallas_call(
        _norm_body,
        grid=(M // MB,),
        in_specs=[pl.BlockSpec((MB, D), lambda i: (i, 0)),
                  pl.BlockSpec((1, D), lambda i: (0, 0)),
                  pl.BlockSpec((1, D), lambda i: (0, 0))],
        out_specs=pl.BlockSpec((MB, D), lambda i: (i, 0)),
        out_shape=jax.ShapeDtypeStruct((M, D), jnp.float32),
    )(mem_raw, g2d, b2d)


SROWS = 784


def _ssm_body(s_ref, out_ref):
    sv = s_ref[...]
    fi = (lax.broadcasted_iota(jnp.int32, (SROWS, D), 0) * D
          + lax.broadcasted_iota(jnp.int32, (SROWS, D), 1))
    valid = fi < M
    a = jnp.abs(sv)
    mx = jnp.max(jnp.where(valid, a, -jnp.inf))
    e = jnp.where(valid, jnp.exp(a - mx), 0.0)
    out_ref[...] = jnp.sign(sv) * (e / jnp.sum(e))


def _state_softmax(state_pad):
    return pl.pallas_call(
        _ssm_body,
        out_shape=jax.ShapeDtypeStruct((SROWS, D), jnp.float32),
    )(state_pad)


def kernel(x, idx, init_state, init_val, ln_gamma, ln_beta):
    idx = idx.astype(jnp.int32)
    idx2d = idx.reshape(N // GB, GB)
    rows, svals = _gather(init_val, init_state, idx2d)
    ww, asc = _route(rows, x, svals.reshape(B, K))
    mem_raw, state_acc = _scatter(x, idx.reshape(N), ww.reshape(N),
                                  asc.reshape(N), init_val, init_state)
    mem = _norm(mem_raw, ln_gamma.reshape(1, D), ln_beta.reshape(1, D))
    state_pad = jnp.pad(state_acc, (0, SROWS * D - M)).reshape(SROWS, D)
    state = _state_softmax(state_pad).reshape(-1)[:M]
    return (mem, state)


# speedup vs baseline: 1.0759x; 1.0759x over previous
import jax
import jax.numpy as jnp
from jax import lax
from jax.experimental import pallas as pl
from jax.experimental.pallas import tpu as pltpu
from jax.experimental.pallas import tpu_sc as plsc

M = 100000
D = 128
B = 16384
K = 8
N = B * K
NC = 2
NS = 16
NW = NC * NS

CH = 10240
NQ = 10
SP = NQ * CH
NSORT = N + 256 * NQ + 512

_MESH = dict(core_axis_name="c", subcore_axis_name="s", num_cores=NC,
             num_subcores=NS)

NPW = N // NW
GB = 128
GI = NPW // GB


def _gather_body(val_hbm, state_hbm, idx_hbm, rows_out, svals_out,
                 idx_v, rows_v, svals_v, sem):
    c = lax.axis_index("c")
    s = lax.axis_index("s")
    wid = s * NC + c
    base = wid * NPW
    pltpu.sync_copy(idx_hbm.at[pl.ds(wid * GI, GI)], idx_v)

    def step(j, carry):
        pltpu.async_copy(val_hbm.at[idx_v.at[j]], rows_v, sem).wait()
        pltpu.async_copy(state_hbm.at[idx_v.at[j]], svals_v, sem).wait()
        pltpu.sync_copy(rows_v, rows_out.at[pl.ds(base + j * GB, GB)])
        pltpu.sync_copy(svals_v, svals_out.at[pl.ds(base + j * GB, GB)])
        return carry

    lax.fori_loop(0, GI, step, 0)


def _gather(init_val, init_state, idx2d):
    return pl.kernel(
        _gather_body,
        out_type=[jax.ShapeDtypeStruct((N, D), jnp.float32),
                  jax.ShapeDtypeStruct((N,), jnp.float32)],
        mesh=plsc.VectorSubcoreMesh(**_MESH),
        scratch_types=[pltpu.VMEM((GI, GB), jnp.int32),
                       pltpu.VMEM((GB, D), jnp.float32),
                       pltpu.VMEM((GB,), jnp.float32),
                       pltpu.SemaphoreType.DMA],
    )(init_val, init_state, idx2d)


RB = 512


def _route_body(rows_ref, x_ref, sv_ref, ww_ref, asc_ref):
    xr = x_ref[...].astype(jnp.bfloat16).astype(jnp.float32)
    r3 = rows_ref[...].astype(jnp.bfloat16).astype(jnp.float32)
    r3 = r3.reshape(RB, K, D)
    sc = jnp.sum(r3 * xr[:, None, :], axis=-1)
    a = jnp.abs(sc)
    e = jnp.exp(a - jnp.max(a, axis=-1, keepdims=True))
    w = jnp.sign(sc) * e / jnp.sum(e, axis=-1, keepdims=True)
    gate = jax.nn.softplus(sv_ref[...])
    ww_ref[...] = w * gate
    asc_ref[...] = a


def _route(rows, x, sv2d):
    return pl.pallas_call(
        _route_body,
        grid=(B // RB,),
        in_specs=[pl.BlockSpec((RB * K, D), lambda i: (i, 0)),
                  pl.BlockSpec((RB, D), lambda i: (i, 0)),
                  pl.BlockSpec((RB, K), lambda i: (i, 0))],
        out_specs=[pl.BlockSpec((RB, K), lambda i: (i, 0)),
                   pl.BlockSpec((RB, K), lambda i: (i, 0))],
        out_shape=[jax.ShapeDtypeStruct((B, K), jnp.float32),
                   jax.ShapeDtypeStruct((B, K), jnp.float32)],
    )(rows, x, sv2d)


PB = 8
NBLK = N // (PB * 128)


def _hist_body(idx_ref, cnt_ref, acc):
    i = pl.program_id(0)

    @pl.when(i == 0)
    def _():
        acc[...] = jnp.zeros_like(acc)

    q = idx_ref[...] // CH
    lane = lax.broadcasted_iota(jnp.int32, (1, 128), 1)
    contrib = jnp.zeros((1, 128), jnp.float32)
    for cdx in range(NQ):
        cc = jnp.sum((q == cdx).astype(jnp.float32))
        contrib = contrib + cc * (lane == cdx).astype(jnp.float32)
    acc[...] = acc[...] + contrib

    @pl.when(i == pl.num_programs(0) - 1)
    def _():
        cnt_ref[...] = acc[...]


def _hist(idx2d):
    return pl.pallas_call(
        _hist_body,
        grid=(NBLK,),
        in_specs=[pl.BlockSpec((PB, 128), lambda i: (i, 0))],
        out_specs=pl.BlockSpec((1, 128), lambda i: (0, 0)),
        out_shape=jax.ShapeDtypeStruct((1, 128), jnp.float32),
        scratch_shapes=[pltpu.VMEM((1, 128), jnp.float32)],
    )(idx2d)


def _pos_body(idx_ref, cnt_ref, dest_ref, pay_ref, bases_ref, counts_ref,
              run_smem):
    i = pl.program_id(0)
    f32 = jnp.float32
    i32 = jnp.int32

    @pl.when(i == 0)
    def _():
        cnt = cnt_ref[...]
        cntr = jnp.floor((cnt + 255.0) / 256.0) * 256.0
        a = lax.broadcasted_iota(i32, (128, 128), 0)
        b = lax.broadcasted_iota(i32, (128, 128), 1)
        us = (a < b).astype(f32)
        bases = jnp.dot(cntr, us, preferred_element_type=f32)
        bases_ref[...] = bases.astype(i32)
        counts_ref[...] = cnt.astype(i32)
        for cdx in range(NQ):
            run_smem[0, cdx] = jnp.sum(bases[0:1, cdx:cdx + 1])

    idxv = idx_ref[...]
    q = idxv // CH
    ai = lax.broadcasted_iota(i32, (128, 128), 0)
    bi = lax.broadcasted_iota(i32, (128, 128), 1)
    uincl = (ai <= bi).astype(f32)
    ri = lax.broadcasted_iota(i32, (PB, PB), 0)
    ci = lax.broadcasted_iota(i32, (PB, PB), 1)
    lstrict = (ci < ri).astype(f32)
    destf = jnp.zeros((PB, 128), f32)
    for cdx in range(NQ):
        m = (q == cdx).astype(f32)
        lcs = jnp.dot(m, uincl, preferred_element_type=f32)
        rowt = lcs[:, 127:128]
        rpe = jnp.dot(lstrict, rowt, preferred_element_type=f32)
        rank = rpe + lcs - m
        runc = run_smem[0, cdx]
        destf = jnp.where(m > 0, runc + rank, destf)
        run_smem[0, cdx] = runc + jnp.sum(m)
    dest_ref[...] = destf.astype(i32)
    row = lax.broadcasted_iota(i32, (PB, 128), 0)
    col = lax.broadcasted_iota(i32, (PB, 128), 1)
    flat = i * (PB * 128) + row * 128 + col
    bb = flat // K
    li = idxv - q * CH
    pay_ref[...] = li + bb * 16384


def _pos(idx2d, counts):
    return pl.pallas_call(
        _pos_body,
        grid=(NBLK,),
        in_specs=[pl.BlockSpec((PB, 128), lambda i: (i, 0)),
                  pl.BlockSpec((1, 128), lambda i: (0, 0))],
        out_specs=[pl.BlockSpec((PB, 128), lambda i: (i, 0)),
                   pl.BlockSpec((PB, 128), lambda i: (i, 0)),
                   pl.BlockSpec((1, 128), lambda i: (0, 0)),
                   pl.BlockSpec((1, 128), lambda i: (0, 0))],
        out_shape=[jax.ShapeDtypeStruct((N // 128, 128), jnp.int32),
                   jax.ShapeDtypeStruct((N // 128, 128), jnp.int32),
                   jax.ShapeDtypeStruct((1, 128), jnp.int32),
                   jax.ShapeDtypeStruct((1, 128), jnp.int32)],
        scratch_shapes=[pltpu.SMEM((1, 16), jnp.float32)],
    )(idx2d, counts)


RT = CH // NS
LQ = NQ - 1
LROWS = M - LQ * CH
LFT = LROWS // RT
LREM = LROWS - LFT * RT
AR = N // NW // 128


def _apply_body(dest_hbm, pay_hbm, ww_hbm, asc_hbm,
                smeta_out, sww_out, sasc_out,
                dscr, pscr, wscr0, ascr0):
    c = lax.axis_index("c")
    s = lax.axis_index("s")
    wid = s * NC + c
    pltpu.sync_copy(dest_hbm.at[pl.ds(wid * AR, AR)], dscr)
    pltpu.sync_copy(pay_hbm.at[pl.ds(wid * AR, AR)], pscr)
    pltpu.sync_copy(ww_hbm.at[pl.ds(wid * AR, AR)], wscr0)
    pltpu.sync_copy(asc_hbm.at[pl.ds(wid * AR, AR)], ascr0)

    def stepA(j, carry):
        pltpu.sync_copy(pscr.at[j], smeta_out.at[dscr.at[j]])
        pltpu.sync_copy(wscr0.at[j], sww_out.at[dscr.at[j]])
        pltpu.sync_copy(ascr0.at[j], sasc_out.at[dscr.at[j]])
        return carry

    lax.fori_loop(0, AR, stepA, 0)


def _apply(dest2d, pay2d, ww2d, asc2d):
    f32 = jnp.float32
    i32 = jnp.int32
    return pl.kernel(
        _apply_body,
        out_type=[jax.ShapeDtypeStruct((NSORT,), i32),
                  jax.ShapeDtypeStruct((NSORT,), f32),
                  jax.ShapeDtypeStruct((NSORT,), f32)],
        mesh=plsc.VectorSubcoreMesh(**_MESH),
        scratch_types=[pltpu.VMEM((AR, 128), i32),
                       pltpu.VMEM((AR, 128), i32),
                       pltpu.VMEM((AR, 128), f32),
                       pltpu.VMEM((AR, 128), f32)],
    )(dest2d, pay2d, ww2d, asc2d)


def _scatter_body(x_hbm, smeta, sww, sasc, basesv, countsv,
                  val_hbm, state_hbm,
                  mem_out, state_out,
                  bscr, cscr, rbuf,
                  mscr, wscr, sascr, xstage, ustage, astage,
                  chunk, schunk, sem):
    c = lax.axis_index("c")
    s = lax.axis_index("s")
    i32 = jnp.int32

    def lane_at(vec16, qq):
        sel = jnp.where(lax.iota(i32, 16) == qq, vec16, 0)
        rbuf[pl.ds(16, 16)] = jnp.zeros((16,), i32)
        red = sel
        for sh in (1, 2, 4, 8):
            rbuf[pl.ds(0, 16)] = red
            red = red + rbuf[pl.ds(sh, 16)]
        return red[0]

    pltpu.sync_copy(basesv, bscr)
    pltpu.sync_copy(countsv, cscr)
    plsc.subcore_barrier()

    for p in range(NQ // NC):
        qq = NC * p + c
        bv = bscr[pl.ds(0, 16)]
        cv = cscr[pl.ds(0, 16)]
        lo = pl.multiple_of(lane_at(bv, qq), 256)
        cntq = lane_at(cv, qq)
        base = pl.multiple_of(qq * CH, 128)

        @pl.when(qq < LQ)
        def _():
            pltpu.sync_copy(val_hbm.at[pl.ds(base + s * RT, RT)],
                            chunk.at[pl.ds(s * RT, RT)])

        @pl.when(qq == LQ)
        def _():
            @pl.when(s < LFT)
            def _():
                pltpu.sync_copy(val_hbm.at[pl.ds(base + s * RT, RT)],
                                chunk.at[pl.ds(s * RT, RT)])

            @pl.when(s == LFT)
            def _():
                pltpu.sync_copy(
                    val_hbm.at[pl.ds(base + LFT * RT, LREM)],
                    chunk.at[pl.ds(LFT * RT, LREM)])

        @pl.when(s == 0)
        def _():
            pltpu.sync_copy(state_hbm.at[pl.ds(base, CH)],
                            schunk.at[pl.ds(0, CH)])

        plsc.subcore_barrier()

        nb = (cntq + 15) // 16
        nog = (nb + 15) // 16
        kmax = (nog + NS - 1) // NS

        def outer(k, carry):
            og = k * NS + s

            @pl.when(og < nog)
            def _():
                ooff = pl.multiple_of(lo + og * 256, 16)
                pltpu.sync_copy(smeta.at[pl.ds(ooff, 256)], mscr)
                pltpu.sync_copy(sww.at[pl.ds(ooff, 256)], wscr)
                pltpu.sync_copy(sasc.at[pl.ds(ooff, 256)], sascr)

                def inner(i16, carry2):
                    ebase = og * 256 + i16 * 16
                    vmask = (ebase + lax.iota(i32, 16)) < cntq
                    meta = mscr[pl.ds(i16 * 16, 16)]
                    li16 = jnp.where(vmask, meta & 16383, CH)
                    b16 = jnp.where(vmask, meta >> 14, 0)
                    pltpu.async_copy(x_hbm.at[b16], xstage, sem).wait()
                    w16 = jnp.where(vmask, wscr[pl.ds(i16 * 16, 16)], 0.0)
                    for j in range(16):
                        wws = w16[j]
                        for vv in range(D // 16):
                            ustage[j, pl.ds(vv * 16, 16)] = (
                                xstage[j, pl.ds(vv * 16, 16)] * wws)
                    a16 = jnp.where(vmask, sascr[pl.ds(i16 * 16, 16)], 0.0)
                    astage[pl.ds(0, 16)] = a16
                    pltpu.sync_copy(ustage, chunk.at[li16], add=True)
                    pltpu.sync_copy(astage, schunk.at[li16], add=True)
                    return carry2

                lax.fori_loop(0, 16, inner, 0)

            return carry

        lax.fori_loop(0, kmax, outer, 0)
        plsc.subcore_barrier()

        @pl.when(qq < LQ)
        def _():
            pltpu.sync_copy(chunk.at[pl.ds(s * RT, RT)],
                            mem_out.at[pl.ds(base + s * RT, RT)])

        @pl.when(qq == LQ)
        def _():
            @pl.when(s < LFT)
            def _():
                pltpu.sync_copy(chunk.at[pl.ds(s * RT, RT)],
                                mem_out.at[pl.ds(base + s * RT, RT)])

            @pl.when(s == LFT)
            def _():
                pltpu.sync_copy(chunk.at[pl.ds(LFT * RT, LREM)],
                                mem_out.at[pl.ds(base + LFT * RT, LREM)])

        @pl.when(s == 0)
        def _():
            pltpu.sync_copy(schunk.at[pl.ds(0, CH)],
                            state_out.at[pl.ds(base, CH)])

        plsc.subcore_barrier()


def _scatter(x, smeta, sww, sasc, basesv, countsv, init_val, state_in):
    f32 = jnp.float32
    i32 = jnp.int32
    return pl.kernel(
        _scatter_body,
        out_type=[jax.ShapeDtypeStruct((M, D), f32),
                  jax.ShapeDtypeStruct((SP,), f32)],
        mesh=plsc.VectorSubcoreMesh(**_MESH),
        scratch_types=[pltpu.VMEM((128,), i32),
                       pltpu.VMEM((128,), i32),
                       pltpu.VMEM((32,), i32),
                       pltpu.VMEM((256,), i32),
                       pltpu.VMEM((256,), f32),
                       pltpu.VMEM((256,), f32),
                       pltpu.VMEM((16, D), f32),
                       pltpu.VMEM((16, D), f32),
                       pltpu.VMEM((16,), f32),
                       pltpu.VMEM_SHARED((CH + 16, D), f32),
                       pltpu.VMEM_SHARED((CH + 16,), f32),
                       pltpu.SemaphoreType.DMA],
    )(x, smeta, sww, sasc, basesv, countsv, init_val, state_in)


MB = 2000


def _norm_body(mem_ref, g_ref, b_ref, out_ref):
    v = mem_ref[...]
    mu = jnp.mean(v, axis=-1, keepdims=True)
    dlt = v - mu
    var = jnp.mean(dlt * dlt, axis=-1, keepdims=True)
    ln = dlt * lax.rsqrt(var + 1e-5) * g_ref[...] + b_ref[...]
    nrm = jnp.sqrt(jnp.sum(ln * ln, axis=-1, keepdims=True))
    out_ref[...] = ln / (nrm + 1e-6)


def _norm(mem_raw, g2d, b2d):
    return pl.pallas_call(
        _norm_body,
        grid=(M // MB,),
        in_specs=[pl.BlockSpec((MB, D), lambda i: (i, 0)),
                  pl.BlockSpec((1, D), lambda i: (0, 0)),
                  pl.BlockSpec((1, D), lambda i: (0, 0))],
        out_specs=pl.BlockSpec((MB, D), lambda i: (i, 0)),
        out_shape=jax.ShapeDtypeStruct((M, D), jnp.float32),
    )(mem_raw, g2d, b2d)


SROWS = SP // D


def _ssm_body(s_ref, out_ref):
    sv = s_ref[...]
    fi = (lax.broadcasted_iota(jnp.int32, (SROWS, D), 0) * D
          + lax.broadcasted_iota(jnp.int32, (SROWS, D), 1))
    valid = fi < M
    a = jnp.abs(sv)
    mx = jnp.max(jnp.where(valid, a, -jnp.inf))
    e = jnp.where(valid, jnp.exp(a - mx), 0.0)
    out_ref[...] = jnp.sign(sv) * (e / jnp.sum(e))


def _state_softmax(state_pad):
    return pl.pallas_call(
        _ssm_body,
        out_shape=jax.ShapeDtypeStruct((SROWS, D), jnp.float32),
    )(state_pad)


def kernel(x, idx, init_state, init_val, ln_gamma, ln_beta):
    idx = idx.astype(jnp.int32)
    idx2d = idx.reshape(N // 128, 128)
    rows, svals = _gather(init_val, init_state, idx2d)
    ww, asc = _route(rows, x, svals.reshape(B, K))
    counts = _hist(idx2d)
    dest2d, pay2d, basesv, countsv = _pos(idx2d, counts)
    state_in = jnp.pad(init_state, (0, SP - M))
    smeta, sww, sasc = _apply(dest2d, pay2d, ww.reshape(N // 128, 128),
                              asc.reshape(N // 128, 128))
    mem_raw, state_acc = _scatter(
        x, smeta, sww, sasc, basesv.reshape(128),
        countsv.reshape(128), init_val, state_in)
    mem = _norm(mem_raw, ln_gamma.reshape(1, D), ln_beta.reshape(1, D))
    state = _state_softmax(state_acc.reshape(SROWS, D)).reshape(-1)[:M]
    return (mem, state)


# speedup vs baseline: 1.1970x; 1.1125x over previous
import jax
import jax.numpy as jnp
from jax import lax
from jax.experimental import pallas as pl
from jax.experimental.pallas import tpu as pltpu
from jax.experimental.pallas import tpu_sc as plsc

M = 100000
D = 128
B = 16384
K = 8
N = B * K
NC = 2
NS = 16
NW = NC * NS

CH = 10240
NQ = 10
SP = NQ * CH
NSORT = N + 256 * NQ + 512

_MESH = dict(core_axis_name="c", subcore_axis_name="s", num_cores=NC,
             num_subcores=NS)

NPW = N // NW
GB = 128
GI = NPW // GB


def _gather_body(val_hbm, state_hbm, idx_hbm, rows_out, svals_out,
                 idx_v, rows_v, svals_v, sem):
    c = lax.axis_index("c")
    s = lax.axis_index("s")
    wid = s * NC + c
    base = wid * NPW
    pltpu.sync_copy(idx_hbm.at[pl.ds(wid * GI, GI)], idx_v)

    def step(j, carry):
        pltpu.async_copy(val_hbm.at[idx_v.at[j]], rows_v, sem).wait()
        pltpu.async_copy(state_hbm.at[idx_v.at[j]], svals_v, sem).wait()
        pltpu.sync_copy(rows_v, rows_out.at[pl.ds(base + j * GB, GB)])
        pltpu.sync_copy(svals_v, svals_out.at[pl.ds(base + j * GB, GB)])
        return carry

    lax.fori_loop(0, GI, step, 0)


def _gather(init_val, init_state, idx2d):
    return pl.kernel(
        _gather_body,
        out_type=[jax.ShapeDtypeStruct((N, D), jnp.float32),
                  jax.ShapeDtypeStruct((N,), jnp.float32)],
        mesh=plsc.VectorSubcoreMesh(**_MESH),
        scratch_types=[pltpu.VMEM((GI, GB), jnp.int32),
                       pltpu.VMEM((GB, D), jnp.float32),
                       pltpu.VMEM((GB,), jnp.float32),
                       pltpu.SemaphoreType.DMA],
    )(init_val, init_state, idx2d)


RB = 512


def _route_body(rows_ref, x_ref, sv_ref, ww_ref, asc_ref):
    xr = x_ref[...].astype(jnp.bfloat16).astype(jnp.float32)
    r3 = rows_ref[...].astype(jnp.bfloat16).astype(jnp.float32)
    r3 = r3.reshape(RB, K, D)
    sc = jnp.sum(r3 * xr[:, None, :], axis=-1)
    a = jnp.abs(sc)
    e = jnp.exp(a - jnp.max(a, axis=-1, keepdims=True))
    w = jnp.sign(sc) * e / jnp.sum(e, axis=-1, keepdims=True)
    gate = jax.nn.softplus(sv_ref[...])
    ww_ref[...] = w * gate
    asc_ref[...] = a


def _route(rows, x, sv2d):
    return pl.pallas_call(
        _route_body,
        grid=(B // RB,),
        in_specs=[pl.BlockSpec((RB * K, D), lambda i: (i, 0)),
                  pl.BlockSpec((RB, D), lambda i: (i, 0)),
                  pl.BlockSpec((RB, K), lambda i: (i, 0))],
        out_specs=[pl.BlockSpec((RB, K), lambda i: (i, 0)),
                   pl.BlockSpec((RB, K), lambda i: (i, 0))],
        out_shape=[jax.ShapeDtypeStruct((B, K), jnp.float32),
                   jax.ShapeDtypeStruct((B, K), jnp.float32)],
    )(rows, x, sv2d)


PB = 8
NBLK = N // (PB * 128)


def _hist_body(idx_ref, cnt_ref, acc):
    i = pl.program_id(0)

    @pl.when(i == 0)
    def _():
        acc[...] = jnp.zeros_like(acc)

    q = idx_ref[...] // CH
    lane = lax.broadcasted_iota(jnp.int32, (1, 128), 1)
    contrib = jnp.zeros((1, 128), jnp.float32)
    for cdx in range(NQ):
        cc = jnp.sum((q == cdx).astype(jnp.float32))
        contrib = contrib + cc * (lane == cdx).astype(jnp.float32)
    acc[...] = acc[...] + contrib

    @pl.when(i == pl.num_programs(0) - 1)
    def _():
        cnt_ref[...] = acc[...]


def _hist(idx2d):
    return pl.pallas_call(
        _hist_body,
        grid=(NBLK,),
        in_specs=[pl.BlockSpec((PB, 128), lambda i: (i, 0))],
        out_specs=pl.BlockSpec((1, 128), lambda i: (0, 0)),
        out_shape=jax.ShapeDtypeStruct((1, 128), jnp.float32),
        scratch_shapes=[pltpu.VMEM((1, 128), jnp.float32)],
    )(idx2d)


def _pos_body(idx_ref, cnt_ref, dest_ref, pay_ref, bases_ref, counts_ref,
              run_smem):
    i = pl.program_id(0)
    f32 = jnp.float32
    i32 = jnp.int32

    @pl.when(i == 0)
    def _():
        cnt = cnt_ref[...]
        cntr = jnp.floor((cnt + 255.0) / 256.0) * 256.0
        a = lax.broadcasted_iota(i32, (128, 128), 0)
        b = lax.broadcasted_iota(i32, (128, 128), 1)
        us = (a < b).astype(f32)
        bases = jnp.dot(cntr, us, preferred_element_type=f32)
        bases_ref[...] = bases.astype(i32)
        counts_ref[...] = cnt.astype(i32)
        for cdx in range(NQ):
            run_smem[0, cdx] = jnp.sum(bases[0:1, cdx:cdx + 1])

    idxv = idx_ref[...]
    q = idxv // CH
    ai = lax.broadcasted_iota(i32, (128, 128), 0)
    bi = lax.broadcasted_iota(i32, (128, 128), 1)
    uincl = (ai <= bi).astype(f32)
    ri = lax.broadcasted_iota(i32, (PB, PB), 0)
    ci = lax.broadcasted_iota(i32, (PB, PB), 1)
    lstrict = (ci < ri).astype(f32)
    destf = jnp.zeros((PB, 128), f32)
    for cdx in range(NQ):
        m = (q == cdx).astype(f32)
        lcs = jnp.dot(m, uincl, preferred_element_type=f32)
        rowt = lcs[:, 127:128]
        rpe = jnp.dot(lstrict, rowt, preferred_element_type=f32)
        rank = rpe + lcs - m
        runc = run_smem[0, cdx]
        destf = jnp.where(m > 0, runc + rank, destf)
        run_smem[0, cdx] = runc + jnp.sum(m)
    dest_ref[...] = destf.astype(i32)
    row = lax.broadcasted_iota(i32, (PB, 128), 0)
    col = lax.broadcasted_iota(i32, (PB, 128), 1)
    flat = i * (PB * 128) + row * 128 + col
    bb = flat // K
    li = idxv - q * CH
    pay_ref[...] = li + bb * 16384


def _pos(idx2d, counts):
    return pl.pallas_call(
        _pos_body,
        grid=(NBLK,),
        in_specs=[pl.BlockSpec((PB, 128), lambda i: (i, 0)),
                  pl.BlockSpec((1, 128), lambda i: (0, 0))],
        out_specs=[pl.BlockSpec((PB, 128), lambda i: (i, 0)),
                   pl.BlockSpec((PB, 128), lambda i: (i, 0)),
                   pl.BlockSpec((1, 128), lambda i: (0, 0)),
                   pl.BlockSpec((1, 128), lambda i: (0, 0))],
        out_shape=[jax.ShapeDtypeStruct((N // 128, 128), jnp.int32),
                   jax.ShapeDtypeStruct((N // 128, 128), jnp.int32),
                   jax.ShapeDtypeStruct((1, 128), jnp.int32),
                   jax.ShapeDtypeStruct((1, 128), jnp.int32)],
        scratch_shapes=[pltpu.SMEM((1, 16), jnp.float32)],
    )(idx2d, counts)


RT = CH // NS
LQ = NQ - 1
LROWS = M - LQ * CH
LFT = LROWS // RT
LREM = LROWS - LFT * RT
AR = N // NW // 128


def _apply_body(dest_hbm, pay_hbm, ww_hbm, asc_hbm,
                smeta_out, sww_out, sasc_out,
                dscr, pscr, wscr0, ascr0, sem):
    c = lax.axis_index("c")
    s = lax.axis_index("s")
    wid = s * NC + c
    pltpu.sync_copy(dest_hbm.at[pl.ds(wid * AR, AR)], dscr)
    pltpu.sync_copy(pay_hbm.at[pl.ds(wid * AR, AR)], pscr)
    pltpu.sync_copy(ww_hbm.at[pl.ds(wid * AR, AR)], wscr0)
    pltpu.sync_copy(asc_hbm.at[pl.ds(wid * AR, AR)], ascr0)

    def stepA(g, carry):
        for j2 in range(8):
            j = g * 8 + j2
            pltpu.async_copy(pscr.at[j], smeta_out.at[dscr.at[j]], sem)
            pltpu.async_copy(wscr0.at[j], sww_out.at[dscr.at[j]], sem)
            pltpu.async_copy(ascr0.at[j], sasc_out.at[dscr.at[j]], sem)
        for j2 in range(8):
            j = g * 8 + j2
            pltpu.make_async_copy(pscr.at[j], smeta_out.at[dscr.at[j]],
                                  sem).wait()
            pltpu.make_async_copy(wscr0.at[j], sww_out.at[dscr.at[j]],
                                  sem).wait()
            pltpu.make_async_copy(ascr0.at[j], sasc_out.at[dscr.at[j]],
                                  sem).wait()
        return carry

    lax.fori_loop(0, AR // 8, stepA, 0)


def _apply(dest2d, pay2d, ww2d, asc2d):
    f32 = jnp.float32
    i32 = jnp.int32
    return pl.kernel(
        _apply_body,
        out_type=[jax.ShapeDtypeStruct((NSORT,), i32),
                  jax.ShapeDtypeStruct((NSORT,), f32),
                  jax.ShapeDtypeStruct((NSORT,), f32)],
        mesh=plsc.VectorSubcoreMesh(**_MESH),
        scratch_types=[pltpu.VMEM((AR, 128), i32),
                       pltpu.VMEM((AR, 128), i32),
                       pltpu.VMEM((AR, 128), f32),
                       pltpu.VMEM((AR, 128), f32),
                       pltpu.SemaphoreType.DMA],
    )(dest2d, pay2d, ww2d, asc2d)


def _scatter_body(x_hbm, smeta, sww, sasc, basesv, countsv,
                  val_hbm, state_hbm,
                  mem_out, state_out,
                  bscr, cscr, rbuf,
                  mscr, wscr, sascr, xs0, xs1, ustage, astage,
                  chunk, schunk, semg0, semg1):
    c = lax.axis_index("c")
    s = lax.axis_index("s")
    i32 = jnp.int32

    def lane_at(vec16, qq):
        sel = jnp.where(lax.iota(i32, 16) == qq, vec16, 0)
        rbuf[pl.ds(16, 16)] = jnp.zeros((16,), i32)
        red = sel
        for sh in (1, 2, 4, 8):
            rbuf[pl.ds(0, 16)] = red
            red = red + rbuf[pl.ds(sh, 16)]
        return red[0]

    pltpu.sync_copy(basesv, bscr)
    pltpu.sync_copy(countsv, cscr)
    plsc.subcore_barrier()

    for p in range(NQ // NC):
        qq = NC * p + c
        bv = bscr[pl.ds(0, 16)]
        cv = cscr[pl.ds(0, 16)]
        lo = pl.multiple_of(lane_at(bv, qq), 256)
        cntq = lane_at(cv, qq)
        base = pl.multiple_of(qq * CH, 128)

        @pl.when(qq < LQ)
        def _():
            pltpu.sync_copy(val_hbm.at[pl.ds(base + s * RT, RT)],
                            chunk.at[pl.ds(s * RT, RT)])

        @pl.when(qq == LQ)
        def _():
            @pl.when(s < LFT)
            def _():
                pltpu.sync_copy(val_hbm.at[pl.ds(base + s * RT, RT)],
                                chunk.at[pl.ds(s * RT, RT)])

            @pl.when(s == LFT)
            def _():
                pltpu.sync_copy(
                    val_hbm.at[pl.ds(base + LFT * RT, LREM)],
                    chunk.at[pl.ds(LFT * RT, LREM)])

        @pl.when(s == 0)
        def _():
            pltpu.sync_copy(state_hbm.at[pl.ds(base, CH)],
                            schunk.at[pl.ds(0, CH)])

        plsc.subcore_barrier()

        nb = (cntq + 15) // 16
        nog = (nb + 15) // 16
        kmax = (nog + NS - 1) // NS

        def outer(k, carry):
            og = k * NS + s

            @pl.when(og < nog)
            def _():
                ooff = pl.multiple_of(lo + og * 256, 16)
                pltpu.sync_copy(smeta.at[pl.ds(ooff, 256)], mscr)
                pltpu.sync_copy(sww.at[pl.ds(ooff, 256)], wscr)
                pltpu.sync_copy(sasc.at[pl.ds(ooff, 256)], sascr)

                def bvec(t):
                    vm = (og * 256 + t * 16 + lax.iota(i32, 16)) < cntq
                    meta = mscr[pl.ds(t * 16, 16)]
                    return vm, meta

                def issue(t, sembuf, xbuf):
                    @pl.when(t < 16)
                    def _():
                        vm, meta = bvec(t)
                        b16 = jnp.where(vm, meta >> 14, 0)
                        pltpu.async_copy(x_hbm.at[b16], xbuf, sembuf)

                def consume(t, sembuf, xbuf):
                    vm, meta = bvec(t)
                    li16 = jnp.where(vm, meta & 16383, CH)
                    b16 = jnp.where(vm, meta >> 14, 0)
                    pltpu.make_async_copy(x_hbm.at[b16], xbuf, sembuf).wait()
                    w16 = jnp.where(vm, wscr[pl.ds(t * 16, 16)], 0.0)
                    for j in range(16):
                        wws = w16[j]
                        for vv in range(D // 16):
                            ustage[j, pl.ds(vv * 16, 16)] = (
                                xbuf[j, pl.ds(vv * 16, 16)] * wws)
                    a16 = jnp.where(vm, sascr[pl.ds(t * 16, 16)], 0.0)
                    astage[pl.ds(0, 16)] = a16
                    pltpu.sync_copy(ustage, chunk.at[li16], add=True)
                    pltpu.sync_copy(astage, schunk.at[li16], add=True)

                issue(jnp.int32(0), semg0, xs0)

                def inner2(h, carry2):
                    t0 = h * 2
                    issue(t0 + 1, semg1, xs1)
                    consume(t0, semg0, xs0)
                    issue(t0 + 2, semg0, xs0)
                    consume(t0 + 1, semg1, xs1)
                    return carry2

                lax.fori_loop(0, 8, inner2, 0)

            return carry

        lax.fori_loop(0, kmax, outer, 0)
        plsc.subcore_barrier()

        @pl.when(qq < LQ)
        def _():
            pltpu.sync_copy(chunk.at[pl.ds(s * RT, RT)],
                            mem_out.at[pl.ds(base + s * RT, RT)])

        @pl.when(qq == LQ)
        def _():
            @pl.when(s < LFT)
            def _():
                pltpu.sync_copy(chunk.at[pl.ds(s * RT, RT)],
                                mem_out.at[pl.ds(base + s * RT, RT)])

            @pl.when(s == LFT)
            def _():
                pltpu.sync_copy(chunk.at[pl.ds(LFT * RT, LREM)],
                                mem_out.at[pl.ds(base + LFT * RT, LREM)])

        @pl.when(s == 0)
        def _():
            pltpu.sync_copy(schunk.at[pl.ds(0, CH)],
                            state_out.at[pl.ds(base, CH)])

        plsc.subcore_barrier()


def _scatter(x, smeta, sww, sasc, basesv, countsv, init_val, state_in):
    f32 = jnp.float32
    i32 = jnp.int32
    return pl.kernel(
        _scatter_body,
        out_type=[jax.ShapeDtypeStruct((M, D), f32),
                  jax.ShapeDtypeStruct((SP,), f32)],
        mesh=plsc.VectorSubcoreMesh(**_MESH),
        scratch_types=[pltpu.VMEM((128,), i32),
                       pltpu.VMEM((128,), i32),
                       pltpu.VMEM((32,), i32),
                       pltpu.VMEM((256,), i32),
                       pltpu.VMEM((256,), f32),
                       pltpu.VMEM((256,), f32),
                       pltpu.VMEM((16, D), f32),
                       pltpu.VMEM((16, D), f32),
                       pltpu.VMEM((16, D), f32),
                       pltpu.VMEM((16,), f32),
                       pltpu.VMEM_SHARED((CH + 16, D), f32),
                       pltpu.VMEM_SHARED((CH + 16,), f32),
                       pltpu.SemaphoreType.DMA,
                       pltpu.SemaphoreType.DMA],
    )(x, smeta, sww, sasc, basesv, countsv, init_val, state_in)


MB = 2000


def _norm_body(mem_ref, g_ref, b_ref, out_ref):
    v = mem_ref[...]
    mu = jnp.mean(v, axis=-1, keepdims=True)
    dlt = v - mu
    var = jnp.mean(dlt * dlt, axis=-1, keepdims=True)
    ln = dlt * lax.rsqrt(var + 1e-5) * g_ref[...] + b_ref[...]
    nrm = jnp.sqrt(jnp.sum(ln * ln, axis=-1, keepdims=True))
    out_ref[...] = ln / (nrm + 1e-6)


def _norm(mem_raw, g2d, b2d):
    return pl.pallas_call(
        _norm_body,
        grid=(M // MB,),
        in_specs=[pl.BlockSpec((MB, D), lambda i: (i, 0)),
                  pl.BlockSpec((1, D), lambda i: (0, 0)),
                  pl.BlockSpec((1, D), lambda i: (0, 0))],
        out_specs=pl.BlockSpec((MB, D), lambda i: (i, 0)),
        out_shape=jax.ShapeDtypeStruct((M, D), jnp.float32),
    )(mem_raw, g2d, b2d)


SROWS = SP // D


def _ssm_body(s_ref, out_ref):
    sv = s_ref[...]
    fi = (lax.broadcasted_iota(jnp.int32, (SROWS, D), 0) * D
          + lax.broadcasted_iota(jnp.int32, (SROWS, D), 1))
    valid = fi < M
    a = jnp.abs(sv)
    mx = jnp.max(jnp.where(valid, a, -jnp.inf))
    e = jnp.where(valid, jnp.exp(a - mx), 0.0)
    out_ref[...] = jnp.sign(sv) * (e / jnp.sum(e))


def _state_softmax(state_pad):
    return pl.pallas_call(
        _ssm_body,
        out_shape=jax.ShapeDtypeStruct((SROWS, D), jnp.float32),
    )(state_pad)


def kernel(x, idx, init_state, init_val, ln_gamma, ln_beta):
    idx = idx.astype(jnp.int32)
    idx2d = idx.reshape(N // 128, 128)
    rows, svals = _gather(init_val, init_state, idx2d)
    ww, asc = _route(rows, x, svals.reshape(B, K))
    counts = _hist(idx2d)
    dest2d, pay2d, basesv, countsv = _pos(idx2d, counts)
    state_in = jnp.pad(init_state, (0, SP - M))
    smeta, sww, sasc = _apply(dest2d, pay2d, ww.reshape(N // 128, 128),
                              asc.reshape(N // 128, 128))
    mem_raw, state_acc = _scatter(
        x, smeta, sww, sasc, basesv.reshape(128),
        countsv.reshape(128), init_val, state_in)
    mem = _norm(mem_raw, ln_gamma.reshape(1, D), ln_beta.reshape(1, D))
    state = _state_softmax(state_acc.reshape(SROWS, D)).reshape(-1)[:M]
    return (mem, state)


# speedup vs baseline: 1.7426x; 1.4558x over previous
import jax
import jax.numpy as jnp
from jax import lax
from jax.experimental import pallas as pl
from jax.experimental.pallas import tpu as pltpu
from jax.experimental.pallas import tpu_sc as plsc

M = 100000
D = 128
B = 16384
K = 8
N = B * K
NC = 2
NS = 16
NW = NC * NS

CH = 10240
NQ = 10
SP = NQ * CH
NSORT = N + 256 * NQ + 512

_MESH = dict(core_axis_name="c", subcore_axis_name="s", num_cores=NC,
             num_subcores=NS)

NPW = N // NW
GB = 128
GI = NPW // GB


def _gather_body(val_hbm, state_hbm, idx_hbm, rows_out, svals_out,
                 idx_v, rows_v, svals_v, sem):
    c = lax.axis_index("c")
    s = lax.axis_index("s")
    wid = s * NC + c
    base = wid * NPW
    pltpu.sync_copy(idx_hbm.at[pl.ds(wid * GI, GI)], idx_v)

    def step(j, carry):
        pltpu.async_copy(val_hbm.at[idx_v.at[j]], rows_v, sem).wait()
        pltpu.async_copy(state_hbm.at[idx_v.at[j]], svals_v, sem).wait()
        pltpu.sync_copy(rows_v, rows_out.at[pl.ds(base + j * GB, GB)])
        pltpu.sync_copy(svals_v, svals_out.at[pl.ds(base + j * GB, GB)])
        return carry

    lax.fori_loop(0, GI, step, 0)


def _gather(init_val, init_state, idx2d):
    return pl.kernel(
        _gather_body,
        out_type=[jax.ShapeDtypeStruct((N, D), jnp.float32),
                  jax.ShapeDtypeStruct((N,), jnp.float32)],
        mesh=plsc.VectorSubcoreMesh(**_MESH),
        scratch_types=[pltpu.VMEM((GI, GB), jnp.int32),
                       pltpu.VMEM((GB, D), jnp.float32),
                       pltpu.VMEM((GB,), jnp.float32),
                       pltpu.SemaphoreType.DMA],
    )(init_val, init_state, idx2d)


RB = 512


def _route_body(rows_ref, x_ref, sv_ref, ww_ref, asc_ref):
    xr = x_ref[...].astype(jnp.bfloat16).astype(jnp.float32)
    r3 = rows_ref[...].astype(jnp.bfloat16).astype(jnp.float32)
    r3 = r3.reshape(RB, K, D)
    sc = jnp.sum(r3 * xr[:, None, :], axis=-1)
    a = jnp.abs(sc)
    e = jnp.exp(a - jnp.max(a, axis=-1, keepdims=True))
    w = jnp.sign(sc) * e / jnp.sum(e, axis=-1, keepdims=True)
    gate = jax.nn.softplus(sv_ref[...])
    ww_ref[...] = w * gate
    asc_ref[...] = a


def _route(rows, x, sv2d):
    return pl.pallas_call(
        _route_body,
        grid=(B // RB,),
        in_specs=[pl.BlockSpec((RB * K, D), lambda i: (i, 0)),
                  pl.BlockSpec((RB, D), lambda i: (i, 0)),
                  pl.BlockSpec((RB, K), lambda i: (i, 0))],
        out_specs=[pl.BlockSpec((RB, K), lambda i: (i, 0)),
                   pl.BlockSpec((RB, K), lambda i: (i, 0))],
        out_shape=[jax.ShapeDtypeStruct((B, K), jnp.float32),
                   jax.ShapeDtypeStruct((B, K), jnp.float32)],
    )(rows, x, sv2d)


PB = 8
NBLK = N // (PB * 128)


def _hist_body(idx_ref, cnt_ref, acc):
    i = pl.program_id(0)

    @pl.when(i == 0)
    def _():
        acc[...] = jnp.zeros_like(acc)

    q = idx_ref[...] // CH
    lane = lax.broadcasted_iota(jnp.int32, (1, 128), 1)
    contrib = jnp.zeros((1, 128), jnp.float32)
    for cdx in range(NQ):
        cc = jnp.sum((q == cdx).astype(jnp.float32))
        contrib = contrib + cc * (lane == cdx).astype(jnp.float32)
    acc[...] = acc[...] + contrib

    @pl.when(i == pl.num_programs(0) - 1)
    def _():
        cnt_ref[...] = acc[...]


def _hist(idx2d):
    return pl.pallas_call(
        _hist_body,
        grid=(NBLK,),
        in_specs=[pl.BlockSpec((PB, 128), lambda i: (i, 0))],
        out_specs=pl.BlockSpec((1, 128), lambda i: (0, 0)),
        out_shape=jax.ShapeDtypeStruct((1, 128), jnp.float32),
        scratch_shapes=[pltpu.VMEM((1, 128), jnp.float32)],
    )(idx2d)


def _pos_body(idx_ref, cnt_ref, dest_ref, pay_ref, bases_ref, counts_ref,
              run_smem):
    i = pl.program_id(0)
    f32 = jnp.float32
    i32 = jnp.int32

    @pl.when(i == 0)
    def _():
        cnt = cnt_ref[...]
        cntr = jnp.floor((cnt + 255.0) / 256.0) * 256.0
        a = lax.broadcasted_iota(i32, (128, 128), 0)
        b = lax.broadcasted_iota(i32, (128, 128), 1)
        us = (a < b).astype(f32)
        bases = jnp.dot(cntr, us, preferred_element_type=f32)
        bases_ref[...] = bases.astype(i32)
        counts_ref[...] = cnt.astype(i32)
        for cdx in range(NQ):
            run_smem[0, cdx] = jnp.sum(bases[0:1, cdx:cdx + 1])

    idxv = idx_ref[...]
    q = idxv // CH
    ai = lax.broadcasted_iota(i32, (128, 128), 0)
    bi = lax.broadcasted_iota(i32, (128, 128), 1)
    uincl = (ai <= bi).astype(f32)
    ri = lax.broadcasted_iota(i32, (PB, PB), 0)
    ci = lax.broadcasted_iota(i32, (PB, PB), 1)
    lstrict = (ci < ri).astype(f32)
    destf = jnp.zeros((PB, 128), f32)
    for cdx in range(NQ):
        m = (q == cdx).astype(f32)
        lcs = jnp.dot(m, uincl, preferred_element_type=f32)
        rowt = lcs[:, 127:128]
        rpe = jnp.dot(lstrict, rowt, preferred_element_type=f32)
        rank = rpe + lcs - m
        runc = run_smem[0, cdx]
        destf = jnp.where(m > 0, runc + rank, destf)
        run_smem[0, cdx] = runc + jnp.sum(m)
    dest_ref[...] = destf.astype(i32)
    row = lax.broadcasted_iota(i32, (PB, 128), 0)
    col = lax.broadcasted_iota(i32, (PB, 128), 1)
    flat = i * (PB * 128) + row * 128 + col
    li = idxv - q * CH
    pay_ref[...] = li + flat * 16384


def _pos(idx2d, counts):
    return pl.pallas_call(
        _pos_body,
        grid=(NBLK,),
        in_specs=[pl.BlockSpec((PB, 128), lambda i: (i, 0)),
                  pl.BlockSpec((1, 128), lambda i: (0, 0))],
        out_specs=[pl.BlockSpec((PB, 128), lambda i: (i, 0)),
                   pl.BlockSpec((PB, 128), lambda i: (i, 0)),
                   pl.BlockSpec((1, 128), lambda i: (0, 0)),
                   pl.BlockSpec((1, 128), lambda i: (0, 0))],
        out_shape=[jax.ShapeDtypeStruct((N // 128, 128), jnp.int32),
                   jax.ShapeDtypeStruct((N // 128, 128), jnp.int32),
                   jax.ShapeDtypeStruct((1, 128), jnp.int32),
                   jax.ShapeDtypeStruct((1, 128), jnp.int32)],
        scratch_shapes=[pltpu.SMEM((1, 16), jnp.float32)],
    )(idx2d, counts)


RT = CH // NS
LQ = NQ - 1
LROWS = M - LQ * CH
LFT = LROWS // RT
LREM = LROWS - LFT * RT
AR = N // NW // 128


def _apply_body(dest_hbm, pay_hbm, smeta_out, dscr, pscr, sem):
    c = lax.axis_index("c")
    s = lax.axis_index("s")
    wid = s * NC + c
    pltpu.sync_copy(dest_hbm.at[pl.ds(wid * AR, AR)], dscr)
    pltpu.sync_copy(pay_hbm.at[pl.ds(wid * AR, AR)], pscr)

    def stepA(g, carry):
        for j2 in range(8):
            j = g * 8 + j2
            pltpu.async_copy(pscr.at[j], smeta_out.at[dscr.at[j]], sem)
        for j2 in range(8):
            j = g * 8 + j2
            pltpu.make_async_copy(pscr.at[j], smeta_out.at[dscr.at[j]],
                                  sem).wait()
        return carry

    lax.fori_loop(0, AR // 8, stepA, 0)


def _apply(dest2d, pay2d):
    i32 = jnp.int32
    return pl.kernel(
        _apply_body,
        out_type=jax.ShapeDtypeStruct((NSORT,), i32),
        mesh=plsc.VectorSubcoreMesh(**_MESH),
        scratch_types=[pltpu.VMEM((AR, 128), i32),
                       pltpu.VMEM((AR, 128), i32),
                       pltpu.SemaphoreType.DMA],
    )(dest2d, pay2d)


def _scatter_body(x_hbm, smeta, wwf, ascf, basesv, countsv,
                  val_hbm, state_hbm,
                  mem_out, state_out,
                  bscr, cscr, rbuf,
                  mscr, xs0, xs1, wb0, wb1, ab0, ab1, ustage, astage,
                  chunk, schunk, semg0, semg1, semw0, semw1):
    c = lax.axis_index("c")
    s = lax.axis_index("s")
    i32 = jnp.int32

    def lane_at(vec16, qq):
        sel = jnp.where(lax.iota(i32, 16) == qq, vec16, 0)
        rbuf[pl.ds(16, 16)] = jnp.zeros((16,), i32)
        red = sel
        for sh in (1, 2, 4, 8):
            rbuf[pl.ds(0, 16)] = red
            red = red + rbuf[pl.ds(sh, 16)]
        return red[0]

    pltpu.sync_copy(basesv, bscr)
    pltpu.sync_copy(countsv, cscr)
    plsc.subcore_barrier()

    for p in range(NQ // NC):
        qq = NC * p + c
        bv = bscr[pl.ds(0, 16)]
        cv = cscr[pl.ds(0, 16)]
        lo = pl.multiple_of(lane_at(bv, qq), 256)
        cntq = lane_at(cv, qq)
        base = pl.multiple_of(qq * CH, 128)

        @pl.when(qq < LQ)
        def _():
            pltpu.sync_copy(val_hbm.at[pl.ds(base + s * RT, RT)],
                            chunk.at[pl.ds(s * RT, RT)])

        @pl.when(qq == LQ)
        def _():
            @pl.when(s < LFT)
            def _():
                pltpu.sync_copy(val_hbm.at[pl.ds(base + s * RT, RT)],
                                chunk.at[pl.ds(s * RT, RT)])

            @pl.when(s == LFT)
            def _():
                pltpu.sync_copy(
                    val_hbm.at[pl.ds(base + LFT * RT, LREM)],
                    chunk.at[pl.ds(LFT * RT, LREM)])

        @pl.when(s == 0)
        def _():
            pltpu.sync_copy(state_hbm.at[pl.ds(base, CH)],
                            schunk.at[pl.ds(0, CH)])

        plsc.subcore_barrier()

        nb = (cntq + 15) // 16
        nog = (nb + 15) // 16
        kmax = (nog + NS - 1) // NS

        def outer(k, carry):
            og = k * NS + s

            @pl.when(og < nog)
            def _():
                ooff = pl.multiple_of(lo + og * 256, 16)
                pltpu.sync_copy(smeta.at[pl.ds(ooff, 256)], mscr)

                def bvec(t):
                    vm = (og * 256 + t * 16 + lax.iota(i32, 16)) < cntq
                    meta = mscr[pl.ds(t * 16, 16)]
                    u16 = jnp.where(vm, meta >> 14, 0)
                    return vm, meta, u16

                def issue(t, sembuf, xbuf, semw, wbuf, abuf):
                    @pl.when(t < 16)
                    def _():
                        vm, meta, u16 = bvec(t)
                        pltpu.async_copy(x_hbm.at[u16 >> 3], xbuf, sembuf)
                        pltpu.async_copy(wwf.at[u16], wbuf, semw)
                        pltpu.async_copy(ascf.at[u16], abuf, semw)

                def consume(t, sembuf, xbuf, semw, wbuf, abuf):
                    vm, meta, u16 = bvec(t)
                    li16 = jnp.where(vm, meta & 16383, CH)
                    pltpu.make_async_copy(x_hbm.at[u16 >> 3], xbuf,
                                          sembuf).wait()
                    pltpu.make_async_copy(wwf.at[u16], wbuf, semw).wait()
                    pltpu.make_async_copy(ascf.at[u16], abuf, semw).wait()
                    w16 = jnp.where(vm, wbuf[pl.ds(0, 16)], 0.0)
                    for j in range(16):
                        wws = w16[j]
                        for vv in range(D // 16):
                            ustage[j, pl.ds(vv * 16, 16)] = (
                                xbuf[j, pl.ds(vv * 16, 16)] * wws)
                    a16 = jnp.where(vm, abuf[pl.ds(0, 16)], 0.0)
                    astage[pl.ds(0, 16)] = a16
                    pltpu.sync_copy(ustage, chunk.at[li16], add=True)
                    pltpu.sync_copy(astage, schunk.at[li16], add=True)

                issue(jnp.int32(0), semg0, xs0, semw0, wb0, ab0)

                def inner2(h, carry2):
                    t0 = h * 2
                    issue(t0 + 1, semg1, xs1, semw1, wb1, ab1)
                    consume(t0, semg0, xs0, semw0, wb0, ab0)
                    issue(t0 + 2, semg0, xs0, semw0, wb0, ab0)
                    consume(t0 + 1, semg1, xs1, semw1, wb1, ab1)
                    return carry2

                lax.fori_loop(0, 8, inner2, 0)

            return carry

        lax.fori_loop(0, kmax, outer, 0)
        plsc.subcore_barrier()

        @pl.when(qq < LQ)
        def _():
            pltpu.sync_copy(chunk.at[pl.ds(s * RT, RT)],
                            mem_out.at[pl.ds(base + s * RT, RT)])

        @pl.when(qq == LQ)
        def _():
            @pl.when(s < LFT)
            def _():
                pltpu.sync_copy(chunk.at[pl.ds(s * RT, RT)],
                                mem_out.at[pl.ds(base + s * RT, RT)])

            @pl.when(s == LFT)
            def _():
                pltpu.sync_copy(chunk.at[pl.ds(LFT * RT, LREM)],
                                mem_out.at[pl.ds(base + LFT * RT, LREM)])

        @pl.when(s == 0)
        def _():
            pltpu.sync_copy(schunk.at[pl.ds(0, CH)],
                            state_out.at[pl.ds(base, CH)])

        plsc.subcore_barrier()


def _scatter(x, smeta, wwf, ascf, basesv, countsv, init_val, state_in):
    f32 = jnp.float32
    i32 = jnp.int32
    return pl.kernel(
        _scatter_body,
        out_type=[jax.ShapeDtypeStruct((M, D), f32),
                  jax.ShapeDtypeStruct((SP,), f32)],
        mesh=plsc.VectorSubcoreMesh(**_MESH),
        scratch_types=[pltpu.VMEM((128,), i32),
                       pltpu.VMEM((128,), i32),
                       pltpu.VMEM((32,), i32),
                       pltpu.VMEM((256,), i32),
                       pltpu.VMEM((16, D), f32),
                       pltpu.VMEM((16, D), f32),
                       pltpu.VMEM((16,), f32),
                       pltpu.VMEM((16,), f32),
                       pltpu.VMEM((16,), f32),
                       pltpu.VMEM((16,), f32),
                       pltpu.VMEM((16, D), f32),
                       pltpu.VMEM((16,), f32),
                       pltpu.VMEM_SHARED((CH + 16, D), f32),
                       pltpu.VMEM_SHARED((CH + 16,), f32),
                       pltpu.SemaphoreType.DMA,
                       pltpu.SemaphoreType.DMA,
                       pltpu.SemaphoreType.DMA,
                       pltpu.SemaphoreType.DMA],
    )(x, smeta, wwf, ascf, basesv, countsv, init_val, state_in)


MB = 2000


def _norm_body(mem_ref, g_ref, b_ref, out_ref):
    v = mem_ref[...]
    mu = jnp.mean(v, axis=-1, keepdims=True)
    dlt = v - mu
    var = jnp.mean(dlt * dlt, axis=-1, keepdims=True)
    ln = dlt * lax.rsqrt(var + 1e-5) * g_ref[...] + b_ref[...]
    nrm = jnp.sqrt(jnp.sum(ln * ln, axis=-1, keepdims=True))
    out_ref[...] = ln / (nrm + 1e-6)


def _norm(mem_raw, g2d, b2d):
    return pl.pallas_call(
        _norm_body,
        grid=(M // MB,),
        in_specs=[pl.BlockSpec((MB, D), lambda i: (i, 0)),
                  pl.BlockSpec((1, D), lambda i: (0, 0)),
                  pl.BlockSpec((1, D), lambda i: (0, 0))],
        out_specs=pl.BlockSpec((MB, D), lambda i: (i, 0)),
        out_shape=jax.ShapeDtypeStruct((M, D), jnp.float32),
    )(mem_raw, g2d, b2d)


SROWS = SP // D


def _ssm_body(s_ref, out_ref):
    sv = s_ref[...]
    fi = (lax.broadcasted_iota(jnp.int32, (SROWS, D), 0) * D
          + lax.broadcasted_iota(jnp.int32, (SROWS, D), 1))
    valid = fi < M
    a = jnp.abs(sv)
    mx = jnp.max(jnp.where(valid, a, -jnp.inf))
    e = jnp.where(valid, jnp.exp(a - mx), 0.0)
    out_ref[...] = jnp.sign(sv) * (e / jnp.sum(e))


def _state_softmax(state_pad):
    return pl.pallas_call(
        _ssm_body,
        out_shape=jax.ShapeDtypeStruct((SROWS, D), jnp.float32),
    )(state_pad)


def kernel(x, idx, init_state, init_val, ln_gamma, ln_beta):
    idx = idx.astype(jnp.int32)
    idx2d = idx.reshape(N // 128, 128)
    rows, svals = _gather(init_val, init_state, idx2d)
    ww, asc = _route(rows, x, svals.reshape(B, K))
    counts = _hist(idx2d)
    dest2d, pay2d, basesv, countsv = _pos(idx2d, counts)
    state_in = jnp.pad(init_state, (0, SP - M))
    smeta = _apply(dest2d, pay2d)
    mem_raw, state_acc = _scatter(
        x, smeta, ww.reshape(N), asc.reshape(N), basesv.reshape(128),
        countsv.reshape(128), init_val, state_in)
    mem = _norm(mem_raw, ln_gamma.reshape(1, D), ln_beta.reshape(1, D))
    state = _state_softmax(state_acc.reshape(SROWS, D)).reshape(-1)[:M]
    return (mem, state)
